# baseline (speedup 1.0000x reference)
import jax
import jax.numpy as jnp
from jax.experimental import pallas as pl
from jax.experimental.pallas import tpu as pltpu


def _body(x_ref, o_ref, sem):
    copy = pltpu.make_async_copy(x_ref, o_ref, sem)
    copy.start()
    copy.wait()


def kernel(inputs):
    m, n = inputs.shape
    out = pl.pallas_call(
        _body,
        in_specs=[pl.BlockSpec(memory_space=pltpu.MemorySpace.HBM)],
        out_specs=pl.BlockSpec(memory_space=pltpu.MemorySpace.HBM),
        out_shape=jax.ShapeDtypeStruct((m, n), inputs.dtype),
        scratch_shapes=[pltpu.SemaphoreType.DMA],
    )(inputs)
    return out


# speedup vs baseline: 41.9775x; 41.9775x over previous
import jax
import jax.numpy as jnp
from jax.experimental import pallas as pl


_ROWS_PER_BLOCK = 32


def _body(x_ref, o_ref):
    o_ref[...] = x_ref[...]


def kernel(inputs):
    m, n = inputs.shape
    rb = _ROWS_PER_BLOCK
    grid = (m // rb,)
    out = pl.pallas_call(
        _body,
        grid=grid,
        in_specs=[pl.BlockSpec((rb, n), lambda i: (i, 0))],
        out_specs=pl.BlockSpec((rb, n), lambda i: (i, 0)),
        out_shape=jax.ShapeDtypeStruct((m, n), inputs.dtype),
    )(inputs)
    return out
